# baseline (speedup 1.0000x reference)
import functools

import jax
import jax.numpy as jnp
from jax import lax
from jax.experimental import pallas as pl
from jax.experimental.pallas import tpu as pltpu
from jax.experimental.pallas import tpu_sc as plsc

_D = 32
_NB = 64
_SP = 56


def _emb_body(pb, seq, ids_hbm, table_hbm, out_hbm, idx_v, rows_v, sem):
    nc = plsc.get_sparse_core_info().num_cores
    wid = lax.axis_index("s") * nc + lax.axis_index("c")
    base = wid * pb
    nchunks = pb // _NB

    def step(i, carry):
        b0 = base + i * _NB
        pltpu.sync_copy(ids_hbm.at[pl.ds(b0, _NB), pl.ds(0, _SP)], idx_v)
        copies = [
            pltpu.async_copy(table_hbm.at[idx_v.at[j]], rows_v.at[j], sem)
            for j in range(_NB)
        ]
        for c in copies:
            c.wait()
        pltpu.sync_copy(
            rows_v.at[:, pl.ds(0, seq), :], out_hbm.at[pl.ds(b0, _NB)]
        )
        return carry

    lax.fori_loop(0, nchunks, step, 0)


def kernel(token_ids, table):
    B, S = token_ids.shape

    ids128 = jnp.pad(token_ids, ((0, 0), (0, 128 - S)), mode="wrap")

    info = plsc.get_sparse_core_info()
    nw = info.num_cores * info.num_subcores
    pb = B // nw

    mesh = plsc.VectorSubcoreMesh(core_axis_name="c", subcore_axis_name="s")
    k = functools.partial(
        pl.kernel,
        mesh=mesh,
        out_type=jax.ShapeDtypeStruct((B, S, _D), jnp.float32),
        scratch_types=[
            pltpu.VMEM((_NB, _SP), jnp.int32),
            pltpu.VMEM((_NB, _SP, _D), jnp.float32),
            pltpu.SemaphoreType.DMA,
        ],
        compiler_params=pltpu.CompilerParams(use_tc_tiling_on_sc=False),
    )(functools.partial(_emb_body, pb, S))

    return k(ids128, table)


# speedup vs baseline: 1.0056x; 1.0056x over previous
import functools

import jax
import jax.numpy as jnp
from jax import lax
from jax.experimental import pallas as pl
from jax.experimental.pallas import tpu as pltpu
from jax.experimental.pallas import tpu_sc as plsc

_D = 32
_NB = 64


def _emb_body(pb, ids_hbm, table_hbm, out_hbm, idx_v, rows_v, sem):
    nc = plsc.get_sparse_core_info().num_cores
    wid = lax.axis_index("s") * nc + lax.axis_index("c")
    base = wid * pb
    nchunks = pb // _NB

    def step(i, carry):
        b0 = base + i * _NB
        pltpu.sync_copy(ids_hbm.at[pl.ds(b0, _NB), :], idx_v)
        copies = [
            pltpu.async_copy(table_hbm.at[idx_v.at[j]], rows_v.at[j], sem)
            for j in range(_NB)
        ]
        for c in copies:
            c.wait()
        pltpu.sync_copy(rows_v, out_hbm.at[pl.ds(b0, _NB)])
        return carry

    lax.fori_loop(0, nchunks, step, 0)


def kernel(token_ids, table):
    B, S = token_ids.shape

    info = plsc.get_sparse_core_info()
    nw = info.num_cores * info.num_subcores
    pb = B // nw

    mesh = plsc.VectorSubcoreMesh(core_axis_name="c", subcore_axis_name="s")
    k = functools.partial(
        pl.kernel,
        mesh=mesh,
        out_type=jax.ShapeDtypeStruct((B, S, _D), jnp.float32),
        scratch_types=[
            pltpu.VMEM((_NB, S), jnp.int32),
            pltpu.VMEM((_NB, S, _D), jnp.float32),
            pltpu.SemaphoreType.DMA,
        ],
        compiler_params=pltpu.CompilerParams(use_tc_tiling_on_sc=False),
    )(functools.partial(_emb_body, pb))

    return k(token_ids, table)
